# baseline (speedup 1.0000x reference)
import functools

import jax
import jax.numpy as jnp
from jax import lax
from jax.experimental import pallas as pl
from jax.experimental.pallas import tpu as pltpu
from jax.experimental.pallas import tpu_sc as plsc

_B = 16
_C = 256
_F = 250
_H = 4
_O = 250
_K = 8
_HB = _B // 2
_R = _HB * _C
_NEG = float("-inf")


def _tc_a_kernel(xf_hbm, wh_ref, bias_ref, wph_ref, bp_ref, out_hbm, adj_ref,
                 xbuf, obuf, in_sem, out_sem):
    f32 = jnp.float32

    def in_copy(half):
        return pltpu.make_async_copy(
            xf_hbm.at[pl.ds(half * _R, _R), :], xbuf.at[half],
            in_sem.at[half])

    def out_copy(half):
        return pltpu.make_async_copy(
            obuf.at[half], out_hbm.at[pl.ds(half * _R, _R), :],
            out_sem.at[half])

    in_copy(0).start()
    in_copy(1).start()

    wc = jnp.zeros((_O, _F), f32)
    bvec = jnp.zeros((1, _O), f32)
    for hd in range(_H):
        wc = wc + jax.lax.dot_general(
            wph_ref[hd], wh_ref[hd], (((1,), (0,)), ((), ())),
            preferred_element_type=f32)
        bvec = bvec + jax.lax.dot_general(
            bias_ref[hd:hd + 1, :], wph_ref[hd], (((1,), (1,)), ((), ())),
            preferred_element_type=f32)
    add0 = bvec + bp_ref[...]

    acc = jnp.zeros((_C, _C), f32)
    for half in range(2):
        in_copy(half).wait()
        xh = xbuf[half]
        for b in range(_HB):
            xb = xh[b * _C:(b + 1) * _C, :]
            mu = jnp.mean(xb, axis=1, keepdims=True)
            xc = xb - mu
            var = jnp.sum(xc * xc, axis=1, keepdims=True) * (1.0 / (_F - 1))
            xn = xc / (jnp.sqrt(var) + 1e-8)
            acc = acc + jax.lax.dot_general(
                xn, xn, (((1,), (1,)), ((), ())), preferred_element_type=f32)
        ob = jax.lax.dot_general(
            xh, wc, (((1,), (1,)), ((), ())),
            preferred_element_type=f32) + add0
        obuf[half] = ob.astype(jnp.bfloat16)
        out_copy(half).start()
    adj_ref[...] = acc * (1.0 / (_B * _F))
    out_copy(0).wait()
    out_copy(1).wait()


def _sc_thr_kernel(adj_hbm, thr_hbm, row_v, thr_v):
    wid = lax.axis_index("s") * 2 + lax.axis_index("c")

    @pl.when(wid < 16)
    def _work():
        base = wid * 16
        lane = lax.iota(jnp.int32, 16)
        trow = jnp.zeros((16,), jnp.float32)
        for r in range(16):
            pltpu.sync_copy(adj_hbm.at[base + r, :], row_v)
            vs = [row_v[pl.ds(j * 16, 16)] for j in range(16)]

            def allmax(vecs):
                m = vecs[0]
                for v in vecs[1:]:
                    m = jnp.maximum(m, v)
                return jnp.max(m)

            thr = allmax(vs)
            for _ in range(_K - 1):
                vs = [jnp.where(v < thr, v, _NEG) for v in vs]
                thr = allmax(vs)
            trow = jnp.where(lane == r, thr, trow)
        thr_v[...] = trow
        pltpu.sync_copy(thr_v, thr_hbm.at[pl.ds(base, 16)])


def _tc_c_kernel(x0_ref, wh_ref, att_s_ref, att_d_ref, bias_ref, wph_ref,
                 bp_ref, thr_ref, adj_ref, alias_ref, out_ref):
    f32 = jnp.float32
    adj = adj_ref[...]
    mask = jnp.logical_and(adj >= thr_ref[...], adj != 0.0)

    rid = jax.lax.broadcasted_iota(jnp.int32, (_C, _C), 0)
    cid = jax.lax.broadcasted_iota(jnp.int32, (_C, _C), 1)
    eye = rid == cid

    x0 = x0_ref[...]
    final0 = jnp.broadcast_to(bp_ref[...], (_C, _O)).astype(f32)
    for hd in range(_H):
        h0h = jax.lax.dot_general(
            x0, wh_ref[hd], (((1,), (1,)), ((), ())),
            preferred_element_type=f32)
        asc = jax.lax.dot_general(
            h0h, att_s_ref[hd:hd + 1, :], (((1,), (1,)), ((), ())),
            preferred_element_type=f32)
        adt = jax.lax.dot_general(
            att_d_ref[hd:hd + 1, :], h0h, (((1,), (1,)), ((), ())),
            preferred_element_type=f32)
        logit = asc + adt
        logit = jnp.where(logit > 0, logit, 0.2 * logit)
        lmask = jnp.where(mask, logit, _NEG)
        ldiag = jnp.max(jnp.where(eye, logit, _NEG), axis=0, keepdims=True)
        m = jnp.maximum(jnp.max(lmask, axis=0, keepdims=True), ldiag)
        e = jnp.exp(lmask - m)
        es = jnp.exp(ldiag - m)
        denom = jnp.sum(e, axis=0, keepdims=True) + es
        attw = (e + jnp.where(eye, es, 0.0)) / denom
        attn = jax.lax.dot_general(
            attw, h0h, (((0,), (0,)), ((), ())), preferred_element_type=f32)
        final0 = final0 + jax.lax.dot_general(
            attn + bias_ref[hd:hd + 1, :], wph_ref[hd], (((1,), (1,)), ((), ())),
            preferred_element_type=f32)
    out_ref[...] = final0.astype(jnp.bfloat16)


def kernel(x, W, att_src, att_dst, bias, Wp, bp):
    f32, bf16 = jnp.float32, jnp.bfloat16
    xf = x.reshape(_B * _C, _F)
    wh = W.reshape(_H, _O, _F).astype(bf16)
    wph = Wp.reshape(_O, _H, _O).transpose(1, 0, 2).astype(bf16)
    att_s = att_src.reshape(_H, _O)
    att_d = att_dst.reshape(_H, _O)
    bias_h = bias.reshape(_H, _O)
    bp2 = bp.reshape(1, _O)
    vspec = pl.BlockSpec(memory_space=pltpu.VMEM)

    out_a, adj = pl.pallas_call(
        _tc_a_kernel,
        in_specs=[pl.BlockSpec(memory_space=pltpu.HBM),
                  vspec, vspec, vspec, vspec],
        out_specs=[pl.BlockSpec(memory_space=pltpu.HBM), vspec],
        out_shape=[jax.ShapeDtypeStruct((_B * _C, _O), bf16),
                   jax.ShapeDtypeStruct((_C, _C), f32)],
        scratch_shapes=[
            pltpu.VMEM((2, _R, _F), f32),
            pltpu.VMEM((2, _R, _O), bf16),
            pltpu.SemaphoreType.DMA((2,)),
            pltpu.SemaphoreType.DMA((2,)),
        ],
        compiler_params=pltpu.CompilerParams(
            allow_input_fusion=[True] * 5,
            skip_device_barrier=True,
        ),
    )(xf, wh, bias_h, wph, bp2)

    sc_thr = pl.kernel(
        _sc_thr_kernel,
        out_type=jax.ShapeDtypeStruct((_C,), f32),
        mesh=plsc.VectorSubcoreMesh(core_axis_name="c", subcore_axis_name="s"),
        scratch_types=[pltpu.VMEM((_C,), f32), pltpu.VMEM((16,), f32)],
        compiler_params=pltpu.CompilerParams(needs_layout_passes=False),
    )
    thr = sc_thr(adj)

    out = pl.pallas_call(
        _tc_c_kernel,
        grid=(1,),
        in_specs=[pl.BlockSpec((_C, _F), lambda i: (0, 0)),
                  pl.BlockSpec((_H, _O, _F), lambda i: (0, 0, 0)),
                  pl.BlockSpec((_H, _O), lambda i: (0, 0)),
                  pl.BlockSpec((_H, _O), lambda i: (0, 0)),
                  pl.BlockSpec((_H, _O), lambda i: (0, 0)),
                  pl.BlockSpec((_H, _O, _F), lambda i: (0, 0, 0)),
                  pl.BlockSpec((1, _O), lambda i: (0, 0)),
                  pl.BlockSpec((_C, 1), lambda i: (0, 0)),
                  pl.BlockSpec((_C, _C), lambda i: (0, 0)),
                  pl.BlockSpec((_C, _O), lambda i: (0, 0))],
        out_specs=pl.BlockSpec((_C, _O), lambda i: (0, 0)),
        out_shape=jax.ShapeDtypeStruct((_B * _C, _O), bf16),
        input_output_aliases={9: 0},
    )(xf[0:_C], wh, att_s, att_d, bias_h, wph, bp2,
      thr.reshape(_C, 1), adj, out_a)
    return out.astype(f32).reshape(_B, 1, _C, _O)


# speedup vs baseline: 2.1742x; 2.1742x over previous
import jax
import jax.numpy as jnp
from jax.experimental import pallas as pl
from jax.experimental.pallas import tpu as pltpu

_B = 16
_C = 256
_F = 250
_H = 4
_O = 250
_K = 8
_HB = _B // 2
_R = _HB * _C
_NEG = float("-inf")


def _eeg_gat_kernel(xf_hbm, wh_ref, att_s_ref, att_d_ref, bias_ref, wph_ref,
                    bp_ref, out_hbm, xbuf, obuf, in_sem, out_sem):
    f32 = jnp.float32

    def in_copy(half):
        return pltpu.make_async_copy(
            xf_hbm.at[pl.ds(half * _R, _R), :], xbuf.at[half],
            in_sem.at[half])

    def out_copy(half):
        return pltpu.make_async_copy(
            obuf.at[half], out_hbm.at[pl.ds(half * _R, _R), :],
            out_sem.at[half])

    in_copy(0).start()
    in_copy(1).start()

    wc = jnp.zeros((_O, _F), f32)
    bvec = jnp.zeros((1, _O), f32)
    for hd in range(_H):
        wc = wc + jax.lax.dot_general(
            wph_ref[hd], wh_ref[hd], (((1,), (0,)), ((), ())),
            preferred_element_type=f32)
        bvec = bvec + jax.lax.dot_general(
            bias_ref[hd:hd + 1, :], wph_ref[hd], (((1,), (1,)), ((), ())),
            preferred_element_type=f32)
    add0 = bvec + bp_ref[...]

    acc = jnp.zeros((_C, _C), f32)
    for half in range(2):
        in_copy(half).wait()
        xh = xbuf[half]
        for b in range(_HB):
            xb = xh[b * _C:(b + 1) * _C, :]
            mu = jnp.mean(xb, axis=1, keepdims=True)
            xc = xb - mu
            var = jnp.sum(xc * xc, axis=1, keepdims=True) * (1.0 / (_F - 1))
            xn = xc / (jnp.sqrt(var) + 1e-8)
            acc = acc + jax.lax.dot_general(
                xn, xn, (((1,), (1,)), ((), ())), preferred_element_type=f32)
        ob = jax.lax.dot_general(
            xh, wc, (((1,), (1,)), ((), ())),
            preferred_element_type=f32) + add0
        obuf[half] = ob.astype(jnp.bfloat16)
        out_copy(half).start()

    adj = acc * (1.0 / (_B * _F))
    work = adj
    thr = jnp.max(work, axis=1, keepdims=True)
    for _ in range(_K - 1):
        work = jnp.where(work < thr, work, _NEG)
        thr = jnp.max(work, axis=1, keepdims=True)
    mask = jnp.logical_and(adj >= thr, adj != 0.0)

    rid = jax.lax.broadcasted_iota(jnp.int32, (_C, _C), 0)
    cid = jax.lax.broadcasted_iota(jnp.int32, (_C, _C), 1)
    eye = rid == cid

    x0 = xbuf[0, 0:_C, :]
    final0 = jnp.broadcast_to(bp_ref[...], (_C, _O)).astype(f32)
    for hd in range(_H):
        h0h = jax.lax.dot_general(
            x0, wh_ref[hd], (((1,), (1,)), ((), ())),
            preferred_element_type=f32)
        asc = jax.lax.dot_general(
            h0h, att_s_ref[hd:hd + 1, :], (((1,), (1,)), ((), ())),
            preferred_element_type=f32)
        adt = jax.lax.dot_general(
            att_d_ref[hd:hd + 1, :], h0h, (((1,), (1,)), ((), ())),
            preferred_element_type=f32)
        logit = asc + adt
        logit = jnp.where(logit > 0, logit, 0.2 * logit)
        lmask = jnp.where(mask, logit, _NEG)
        ldiag = jnp.max(jnp.where(eye, logit, _NEG), axis=0, keepdims=True)
        m = jnp.maximum(jnp.max(lmask, axis=0, keepdims=True), ldiag)
        e = jnp.exp(lmask - m)
        es = jnp.exp(ldiag - m)
        denom = jnp.sum(e, axis=0, keepdims=True) + es
        attw = (e + jnp.where(eye, es, 0.0)) / denom
        attn = jax.lax.dot_general(
            attw, h0h, (((0,), (0,)), ((), ())), preferred_element_type=f32)
        final0 = final0 + jax.lax.dot_general(
            attn + bias_ref[hd:hd + 1, :], wph_ref[hd], (((1,), (1,)), ((), ())),
            preferred_element_type=f32)

    out_copy(0).wait()
    obuf[0, 0:_C, :] = final0.astype(jnp.bfloat16)
    blk0 = pltpu.make_async_copy(
        obuf.at[0, pl.ds(0, _C), :], out_hbm.at[pl.ds(0, _C), :],
        in_sem.at[0])
    blk0.start()
    blk0.wait()
    out_copy(1).wait()


def kernel(x, W, att_src, att_dst, bias, Wp, bp):
    bf16 = jnp.bfloat16
    xf = x.reshape(_B * _C, _F)
    wh = W.reshape(_H, _O, _F).astype(bf16)
    wph = Wp.reshape(_O, _H, _O).transpose(1, 0, 2).astype(bf16)
    att_s = att_src.reshape(_H, _O)
    att_d = att_dst.reshape(_H, _O)
    bias_h = bias.reshape(_H, _O)
    bp2 = bp.reshape(1, _O)
    vspec = pl.BlockSpec(memory_space=pltpu.VMEM)
    out = pl.pallas_call(
        _eeg_gat_kernel,
        in_specs=[pl.BlockSpec(memory_space=pltpu.HBM),
                  vspec, vspec, vspec, vspec, vspec, vspec],
        out_specs=pl.BlockSpec(memory_space=pltpu.HBM),
        out_shape=jax.ShapeDtypeStruct((_B * _C, _O), bf16),
        scratch_shapes=[
            pltpu.VMEM((2, _R, _F), jnp.float32),
            pltpu.VMEM((2, _R, _O), bf16),
            pltpu.SemaphoreType.DMA((2,)),
            pltpu.SemaphoreType.DMA((2,)),
        ],
        compiler_params=pltpu.CompilerParams(
            allow_input_fusion=[True] * 7,
            skip_device_barrier=True,
        ),
    )(xf, wh, att_s, att_d, bias_h, wph, bp2)
    return out.astype(jnp.float32).reshape(_B, 1, _C, _O)
